# baseline (speedup 1.0000x reference)
import jax
import jax.numpy as jnp
from jax import lax
from jax.experimental import pallas as pl
from jax.experimental.pallas import tpu as pltpu
from jax.experimental.pallas import tpu_sc as plsc

N = 50000
E = 800000
HID = 64
NUM_GRAPHS = 128

NPAD = 50176
NROW = NPAD // 128
NT = 50304
Q = 12544
ACCQ = 12552
RPSQ = Q // 16
EPW = 25088
EPAD = 32 * EPW
CAPQ = 7680
BLK = 512
MBLK = 96
NSLQ = 128
LEAK = 0.01
NEG = -1.0e30

_mesh = plsc.VectorSubcoreMesh(core_axis_name="c", subcore_axis_name="s")
_sc_params = pltpu.CompilerParams(needs_layout_passes=False)


def _f32(shape):
    return jax.ShapeDtypeStruct(shape, jnp.float32)


def _i32(shape):
    return jax.ShapeDtypeStruct(shape, jnp.int32)


def _leaky(v):
    return jnp.where(v >= 0, v, LEAK * v)


def _lane_splat(v, j):
    idx = jnp.full((16, 1), j, dtype=jnp.int32)
    dnums = lax.GatherDimensionNumbers(
        offset_dims=(), collapsed_slice_dims=(0,), start_index_map=(0,))
    return lax.gather(v, idx, dnums, (1,),
                      mode=lax.GatherScatterMode.PROMISE_IN_BOUNDS)


def _scnt(mask):
    return jnp.max(plsc.all_reduce_population_count(mask))


def _qrank(cq, m):
    rank = plsc.cumsum(jnp.where(m, 1.0, 0.0)).astype(jnp.int32)
    return cq + rank - 1


def _part_body(src_h, dst_h, srcp_h, dstp_h, cnt_h,
               src_c, dst_c, sl0, dl0, sl1, dl1, sl2, dl2, sl3, dl3, cntv):
    wid = lax.axis_index("s") * 2 + lax.axis_index("c")
    base = wid * EPW
    lanes = lax.iota(jnp.int32, 16)
    sls = (sl0, sl1, sl2, sl3)
    dls = (dl0, dl1, dl2, dl3)

    def chunk(co, nvr, cnts):
        pltpu.sync_copy(src_h.at[pl.ds(base + co, nvr * 16)],
                        src_c.at[pl.ds(0, nvr * 16)])
        pltpu.sync_copy(dst_h.at[pl.ds(base + co, nvr * 16)],
                        dst_c.at[pl.ds(0, nvr * 16)])

        def vbody(v, cnts):
            sv = src_c[pl.ds(v * 16, 16)]
            dv = dst_c[pl.ds(v * 16, 16)]
            out = []
            for q in range(4):
                m = jnp.logical_and(dv >= q * Q, dv < (q + 1) * Q)
                iq = _qrank(cnts[q], m)
                plsc.store_scatter(sls[q], [iq], sv, mask=m)
                plsc.store_scatter(dls[q], [iq], dv - q * Q, mask=m)
                out.append(cnts[q] + _scnt(m))
            return tuple(out)

        return lax.fori_loop(0, nvr, vbody, cnts)

    cnts = (jnp.int32(0),) * 4
    def outer(k, cnts):
        return chunk(k * 4096, 256, cnts)
    cnts = lax.fori_loop(0, 6, outer, cnts)
    cnts = chunk(6 * 4096, 32, cnts)

    dumdst = Q + (lanes & 7)
    zsrc = jnp.zeros((16,), jnp.int32)
    for q in range(4):
        for i in range(32):
            sls[q][pl.ds(cnts[q] + i * 16, 16)] = zsrc
            dls[q][pl.ds(cnts[q] + i * 16, 16)] = dumdst

    for q in range(4):
        off = (q * 32 + wid) * CAPQ
        pltpu.sync_copy(sls[q], srcp_h.at[pl.ds(off, CAPQ)])
        pltpu.sync_copy(dls[q], dstp_h.at[pl.ds(off, CAPQ)])
        cntv[...] = jnp.broadcast_to(cnts[q], (16,)).astype(jnp.int32)
        pltpu.sync_copy(cntv, cnt_h.at[q * 32 + wid])


_partition = pl.kernel(
    _part_body,
    out_type=(_i32((NSLQ * CAPQ,)), _i32((NSLQ * CAPQ,)), _i32((NSLQ, 16))),
    mesh=_mesh,
    compiler_params=_sc_params,
    scratch_types=[
        pltpu.VMEM((4096,), jnp.int32), pltpu.VMEM((4096,), jnp.int32),
        pltpu.VMEM((CAPQ,), jnp.int32), pltpu.VMEM((CAPQ,), jnp.int32),
        pltpu.VMEM((CAPQ,), jnp.int32), pltpu.VMEM((CAPQ,), jnp.int32),
        pltpu.VMEM((CAPQ,), jnp.int32), pltpu.VMEM((CAPQ,), jnp.int32),
        pltpu.VMEM((CAPQ,), jnp.int32), pltpu.VMEM((CAPQ,), jnp.int32),
        pltpu.VMEM((16,), jnp.int32),
    ],
)


def _apart_body(a_h, dst_h, ap_h,
                a_c, dst_c, al0, al1, al2, al3):
    wid = lax.axis_index("s") * 2 + lax.axis_index("c")
    base = wid * EPW
    als = (al0, al1, al2, al3)

    def chunk(co, nvr, cnts):
        pltpu.sync_copy(a_h.at[pl.ds(base + co, nvr * 16)],
                        a_c.at[pl.ds(0, nvr * 16)])
        pltpu.sync_copy(dst_h.at[pl.ds(base + co, nvr * 16)],
                        dst_c.at[pl.ds(0, nvr * 16)])

        def vbody(v, cnts):
            av = a_c[pl.ds(v * 16, 16)]
            dv = dst_c[pl.ds(v * 16, 16)]
            out = []
            for q in range(4):
                m = jnp.logical_and(dv >= q * Q, dv < (q + 1) * Q)
                iq = _qrank(cnts[q], m)
                plsc.store_scatter(als[q], [iq], av, mask=m)
                out.append(cnts[q] + _scnt(m))
            return tuple(out)

        return lax.fori_loop(0, nvr, vbody, cnts)

    cnts = (jnp.int32(0),) * 4
    def outer(k, cnts):
        return chunk(k * 4096, 256, cnts)
    cnts = lax.fori_loop(0, 6, outer, cnts)
    cnts = chunk(6 * 4096, 32, cnts)

    negv = jnp.full((16,), NEG, jnp.float32)
    for q in range(4):
        for i in range(32):
            als[q][pl.ds(cnts[q] + i * 16, 16)] = negv

    for q in range(4):
        off = (q * 32 + wid) * CAPQ
        pltpu.sync_copy(als[q], ap_h.at[pl.ds(off, CAPQ)])


_apart = pl.kernel(
    _apart_body,
    out_type=_f32((NSLQ * CAPQ,)),
    mesh=_mesh,
    compiler_params=_sc_params,
    scratch_types=[
        pltpu.VMEM((4096,), jnp.float32), pltpu.VMEM((4096,), jnp.int32),
        pltpu.VMEM((CAPQ,), jnp.float32), pltpu.VMEM((CAPQ,), jnp.float32),
        pltpu.VMEM((CAPQ,), jnp.float32), pltpu.VMEM((CAPQ,), jnp.float32),
    ],
)


def _alpha_body(s_h, d_h, src_h, dst_h,
                a_h, amax_h,
                stab, dtab, src_c, dst_c, a_c, maxv):
    wid = lax.axis_index("s") * 2 + lax.axis_index("c")
    base = wid * EPW
    pltpu.sync_copy(s_h, stab.at[pl.ds(0, NPAD)])
    pltpu.sync_copy(d_h, dtab.at[pl.ds(0, NPAD)])
    zf = jnp.zeros((16,), jnp.float32)
    for k in range((NT - NPAD) // 16):
        stab[pl.ds(NPAD + k * 16, 16)] = zf
        dtab[pl.ds(NPAD + k * 16, 16)] = zf

    def chunk(co, nvr, mx):
        pltpu.sync_copy(src_h.at[pl.ds(base + co, nvr * 16)],
                        src_c.at[pl.ds(0, nvr * 16)])
        pltpu.sync_copy(dst_h.at[pl.ds(base + co, nvr * 16)],
                        dst_c.at[pl.ds(0, nvr * 16)])

        def vbody(v, mx):
            sv = src_c[pl.ds(v * 16, 16)]
            dv = dst_c[pl.ds(v * 16, 16)]
            dcl = jnp.minimum(dv, NT - 1)
            a = _leaky(plsc.load_gather(stab, [sv])
                       + plsc.load_gather(dtab, [dcl]))
            a_c[pl.ds(v * 16, 16)] = a
            return jnp.maximum(mx, jnp.where(dv < NPAD, a, NEG))

        mx = lax.fori_loop(0, nvr, vbody, mx)
        pltpu.sync_copy(a_c.at[pl.ds(0, nvr * 16)],
                        a_h.at[pl.ds(base + co, nvr * 16)])
        return mx

    mx = jnp.full((16,), NEG, jnp.float32)
    def outer(k, mx):
        return chunk(k * 4096, 256, mx)
    mx = lax.fori_loop(0, 6, outer, mx)
    mx = chunk(6 * 4096, 32, mx)
    maxv[...] = mx
    pltpu.sync_copy(maxv, amax_h.at[wid])


_alpha = pl.kernel(
    _alpha_body,
    out_type=(_f32((EPAD,)), _f32((32, 16))),
    mesh=_mesh,
    compiler_params=_sc_params,
    scratch_types=[
        pltpu.VMEM((NT,), jnp.float32), pltpu.VMEM((NT,), jnp.float32),
        pltpu.VMEM((4096,), jnp.int32), pltpu.VMEM((4096,), jnp.int32),
        pltpu.VMEM((4096,), jnp.float32), pltpu.VMEM((16,), jnp.float32),
    ],
)


def _galpha_body(u_h, t_h, w2_h, attl_h, src_h, dst_h,
                 ea0_h, ea1_h, ea2_h, ea3_h,
                 a_h, amax_h,
                 ttab, urows, src_c, dst_c, a_c,
                 e0c, e1c, e2c, e3c, w2v, attlv, maxv, sem):
    wid = lax.axis_index("s") * 2 + lax.axis_index("c")
    base = wid * EPW
    pltpu.sync_copy(t_h, ttab.at[pl.ds(0, NPAD)])
    zf = jnp.zeros((16,), jnp.float32)
    for k in range((NT - NPAD) // 16):
        ttab[pl.ds(NPAD + k * 16, 16)] = zf
    pltpu.sync_copy(w2_h, w2v)
    pltpu.sync_copy(attl_h, attlv)
    w2 = [[w2v[pl.ds(k * 64 + c * 16, 16)] for c in range(4)]
          for k in range(4)]
    attl = [attlv[pl.ds(c * 16, 16)] for c in range(4)]
    eacs = (e0c, e1c, e2c, e3c)
    eahs = (ea0_h, ea1_h, ea2_h, ea3_h)

    def blk(b, mx):
        off = base + b * BLK
        pltpu.sync_copy(src_h.at[pl.ds(off, BLK)], src_c)
        pltpu.sync_copy(dst_h.at[pl.ds(off, BLK)], dst_c)
        for k in range(4):
            pltpu.sync_copy(eahs[k].at[pl.ds(off, BLK)], eacs[k])
        pltpu.async_copy(u_h.at[src_c], urows, sem).wait()

        def vbody(v, mx):
            phi = jnp.zeros((16,), jnp.float32)
            eav = [eacs[k][pl.ds(v * 16, 16)] for k in range(4)]
            for j in range(16):
                e = v * 16 + j
                eas = [_lane_splat(eav[k], j) for k in range(4)]
                acc = jnp.zeros((16,), jnp.float32)
                for c in range(4):
                    z = urows[e, pl.ds(c * 16, 16)]
                    for k in range(4):
                        z = z + eas[k] * w2[k][c]
                    acc = acc + _leaky(z) * attl[c]
                sjs = jnp.sum(acc)
                phi = jnp.where(lax.iota(jnp.int32, 16) == j, sjs, phi)
            dv = dst_c[pl.ds(v * 16, 16)]
            dcl = jnp.minimum(dv, NT - 1)
            a = _leaky(phi + plsc.load_gather(ttab, [dcl]))
            a_c[pl.ds(v * 16, 16)] = a
            return jnp.maximum(mx, jnp.where(dv < NPAD, a, NEG))

        mx = lax.fori_loop(0, BLK // 16, vbody, mx)
        pltpu.sync_copy(a_c, a_h.at[pl.ds(off, BLK)])
        return mx

    mx = jnp.full((16,), NEG, jnp.float32)
    mx = lax.fori_loop(0, EPW // BLK, blk, mx)
    maxv[...] = mx
    pltpu.sync_copy(maxv, amax_h.at[wid])


_galpha = pl.kernel(
    _galpha_body,
    out_type=(_f32((EPAD,)), _f32((32, 16))),
    mesh=_mesh,
    compiler_params=_sc_params,
    scratch_types=[
        pltpu.VMEM((NT,), jnp.float32),
        pltpu.VMEM((BLK, 128), jnp.float32),
        pltpu.VMEM((BLK,), jnp.int32), pltpu.VMEM((BLK,), jnp.int32),
        pltpu.VMEM((BLK,), jnp.float32),
        pltpu.VMEM((BLK,), jnp.float32), pltpu.VMEM((BLK,), jnp.float32),
        pltpu.VMEM((BLK,), jnp.float32), pltpu.VMEM((BLK,), jnp.float32),
        pltpu.VMEM((256,), jnp.float32), pltpu.VMEM((64,), jnp.float32),
        pltpu.VMEM((16,), jnp.float32),
        pltpu.SemaphoreType.DMA,
    ],
)


def _make_message(pair):

    def body(m_h, ap_h, amax_h, srcp_h, dstp_h, cnt_h, z_h,
             hagg_h,
             rows0, rows1, sb0, sb1, dblk, ablk, amv, cntv, sem0, sem1,
             acc_sh):
        c = lax.axis_index("c")
        s = lax.axis_index("s")
        qrow = (2 * pair + c) * 32

        pltpu.sync_copy(z_h.at[pl.ds(0, RPSQ)],
                        acc_sh.at[pl.ds(s * RPSQ, RPSQ)])

        @pl.when(s == 15)
        def _():
            pltpu.sync_copy(z_h.at[pl.ds(0, 8)], acc_sh.at[pl.ds(Q, 8)])

        pltpu.sync_copy(amax_h, amv)
        gv = jnp.full((16,), NEG, jnp.float32)
        for w in range(32):
            gv = jnp.maximum(gv, amv[w])
        gmax = jnp.max(gv)
        colm = (lax.iota(jnp.int32, 16) == 0).astype(jnp.float32)

        plsc.subcore_barrier()

        rbufs = (rows0, rows1)
        sbufs = (sb0, sb1)
        sems = (sem0, sem1)

        def load_idx(dst_ref, off):
            pltpu.sync_copy(srcp_h.at[pl.ds(off, MBLK)], dst_ref)
            for vv in range(MBLK // 16):
                sl = dst_ref[pl.ds(vv * 16, 16)]
                dst_ref[pl.ds(vv * 16, 16)] = jnp.clip(sl, 0, NPAD - 1)

        def do_slot(w):
            row = qrow + w
            cnt = _read_cnt(cnt_h, row, cntv)
            nblk = (cnt + (MBLK - 1)) // MBLK
            npair2 = (nblk + 1) >> 1

            base = row * CAPQ
            load_idx(sb0, base)
            pltpu.async_copy(m_h.at[sb0], rows0, sem0)

            def pair_body(i, carry):
                for sub in range(2):
                    b = 2 * i + sub
                    off = base + b * MBLK
                    rows = rbufs[sub]
                    pltpu.make_async_copy(m_h.at[sbufs[sub]], rows,
                                          sems[sub]).wait()
                    load_idx(sbufs[1 - sub], off + MBLK)
                    pltpu.async_copy(m_h.at[sbufs[1 - sub]], rbufs[1 - sub],
                                     sems[1 - sub])
                    pltpu.sync_copy(dstp_h.at[pl.ds(off, MBLK)], dblk)
                    pltpu.sync_copy(ap_h.at[pl.ds(off, MBLK)], ablk)

                    def vbody(v, carry2):
                        ex = jnp.exp(ablk[pl.ds(v * 16, 16)] - gmax)
                        for j in range(16):
                            wj = _lane_splat(ex, j)
                            e = v * 16 + j
                            for ch in range(4):
                                rows[e, pl.ds(ch * 16, 16)] = (
                                    rows[e, pl.ds(ch * 16, 16)] * wj)
                            rows[e, pl.ds(64, 16)] = wj * colm
                        return carry2

                    lax.fori_loop(0, MBLK // 16, vbody, 0)
                    pltpu.sync_copy(rows, acc_sh.at[dblk], add=True)
                return carry

            lax.fori_loop(0, npair2, pair_body, 0)
            pltpu.make_async_copy(m_h.at[sb0], rows0, sem0).wait()

        do_slot(2 * s)
        do_slot(2 * s + 1)

        plsc.subcore_barrier()
        pltpu.sync_copy(acc_sh.at[pl.ds(s * RPSQ, RPSQ)],
                        hagg_h.at[pl.ds(c * Q + s * RPSQ, RPSQ)])

    return pl.kernel(
        body,
        out_type=_f32((2 * Q, 128)),
        mesh=_mesh,
        compiler_params=_sc_params,
        scratch_types=[
            pltpu.VMEM((MBLK, 128), jnp.float32),
            pltpu.VMEM((MBLK, 128), jnp.float32),
            pltpu.VMEM((MBLK,), jnp.int32), pltpu.VMEM((MBLK,), jnp.int32),
            pltpu.VMEM((MBLK,), jnp.int32),
            pltpu.VMEM((MBLK,), jnp.float32),
            pltpu.VMEM((32, 16), jnp.float32), pltpu.VMEM((16,), jnp.int32),
            pltpu.SemaphoreType.DMA, pltpu.SemaphoreType.DMA,
            pltpu.VMEM_SHARED((ACCQ, 128), jnp.float32),
        ],
    )


_message0 = _make_message(0)
_message1 = _make_message(1)


def _read_cnt(cnt_h, row, cntv):
    pltpu.sync_copy(cnt_h.at[row], cntv)
    return jnp.max(cntv[...])


def _dotT(a, b):
    return lax.dot_general(a, b, (((1,), (1,)), ((), ())),
                           preferred_element_type=jnp.float32)


def _rowsum(x):
    return jnp.sum(x, axis=1, keepdims=True)


def _s1mat():
    r = lax.broadcasted_iota(jnp.int32, (8, 1024), 0)
    n = lax.broadcasted_iota(jnp.int32, (8, 1024), 1)
    return (r == (n >> 7)).astype(jnp.float32)


def _m128mat():
    n = lax.broadcasted_iota(jnp.int32, (1024, 128), 0)
    c = lax.broadcasted_iota(jnp.int32, (1024, 128), 1)
    return ((n & 127) == c).astype(jnp.float32)


def _col2plane(tcol):
    return lax.dot_general(_s1mat(), _lb(tcol, 128) * _m128mat(),
                           (((1,), (0,)), ((), ())),
                           preferred_element_type=jnp.float32)


def _lb(col, n):
    return lax.dot_general(col, jnp.ones((1, n), jnp.float32),
                           (((1,), (0,)), ((), ())),
                           preferred_element_type=jnp.float32)


def _plane2col(p8):
    y = lax.dot_general(_s1mat(), p8, (((0,), (0,)), ((), ())),
                        preferred_element_type=jnp.float32)
    return _rowsum(y * _m128mat())


def _gru(h, x, wih, whh):
    gi = _dotT(h, wih)
    gh = _dotT(x, whh)
    r = jax.nn.sigmoid(gi[:, 0:64] + gh[:, 0:64])
    z = jax.nn.sigmoid(gi[:, 64:128] + gh[:, 64:128])
    n = jnp.tanh(gi[:, 128:192] + r * gh[:, 128:192])
    return (1.0 - z) * n + z * x


def _elu(v):
    return jnp.where(v > 0, v, jnp.exp(v) - 1.0)


def _tc0_body(x_ref, w1_ref, b1_ref, wa_ref, w2g_ref, attr_ref,
              x1_ref, u_ref, m_ref, t_ref):
    i = pl.program_id(0)
    x1 = _leaky(_dotT(x_ref[...], w1_ref[...]) + b1_ref[...])
    ridx = i * 1024 + lax.broadcasted_iota(jnp.int32, (1024, 64), 0)
    x1 = jnp.where(ridx < N, x1, 0.0)
    x1_ref[...] = x1
    z = jnp.zeros((1024, 64), jnp.float32)
    u_ref[...] = jnp.concatenate([_dotT(x1, wa_ref[...]), z], axis=1)
    m_ref[...] = jnp.concatenate([_dotT(x1, w2g_ref[...]), z], axis=1)
    t_ref[...] = _col2plane(_rowsum(x1 * attr_ref[...]))


def _tc0(xp, w1p, b1, wa, w2g, attr):
    return pl.pallas_call(
        _tc0_body,
        grid=(49,),
        in_specs=[
            pl.BlockSpec((1024, 32), lambda i: (i, 0)),
            pl.BlockSpec((64, 32), lambda i: (0, 0)),
            pl.BlockSpec((1, 64), lambda i: (0, 0)),
            pl.BlockSpec((64, 64), lambda i: (0, 0)),
            pl.BlockSpec((64, 64), lambda i: (0, 0)),
            pl.BlockSpec((1, 64), lambda i: (0, 0)),
        ],
        out_specs=[
            pl.BlockSpec((1024, 64), lambda i: (i, 0)),
            pl.BlockSpec((1024, 128), lambda i: (i, 0)),
            pl.BlockSpec((1024, 128), lambda i: (i, 0)),
            pl.BlockSpec((8, 128), lambda i: (i, 0)),
        ],
        out_shape=[_f32((NPAD, 64)), _f32((NPAD, 128)), _f32((NPAD, 128)),
                   _f32((NROW, 128))],
    )(xp, w1p, b1, wa, w2g, attr)


def _tclayer_body(hagg_ref, x_ref, bias_ref, wih_ref, whh_ref,
                  wn_ref, asrc_ref, adst_ref,
                  xn_ref, xp_ref, s_ref, d_ref):
    hb = hagg_ref[...]
    den = _lb(jnp.maximum(hb[:, 64:65], 1e-30), 64)
    h = _elu(hb[:, 0:64] / den + bias_ref[...])
    xn = jnp.maximum(_gru(h, x_ref[...], wih_ref[...], whh_ref[...]), 0.0)
    xn_ref[...] = xn
    xp = _dotT(xn, wn_ref[...])
    xp_ref[...] = jnp.concatenate(
        [xp, jnp.zeros((1024, 64), jnp.float32)], axis=1)
    s_ref[...] = _col2plane(_rowsum(xp * asrc_ref[...]))
    d_ref[...] = _col2plane(_rowsum(xp * adst_ref[...]))


def _tclayer(hagg, x, bias, wih, whh, wn, asrc, adst):
    return pl.pallas_call(
        _tclayer_body,
        grid=(49,),
        in_specs=[
            pl.BlockSpec((1024, 128), lambda i: (i, 0)),
            pl.BlockSpec((1024, 64), lambda i: (i, 0)),
            pl.BlockSpec((1, 64), lambda i: (0, 0)),
            pl.BlockSpec((192, 64), lambda i: (0, 0)),
            pl.BlockSpec((192, 64), lambda i: (0, 0)),
            pl.BlockSpec((64, 64), lambda i: (0, 0)),
            pl.BlockSpec((1, 64), lambda i: (0, 0)),
            pl.BlockSpec((1, 64), lambda i: (0, 0)),
        ],
        out_specs=[
            pl.BlockSpec((1024, 64), lambda i: (i, 0)),
            pl.BlockSpec((1024, 128), lambda i: (i, 0)),
            pl.BlockSpec((8, 128), lambda i: (i, 0)),
            pl.BlockSpec((8, 128), lambda i: (i, 0)),
        ],
        out_shape=[_f32((NPAD, 64)), _f32((NPAD, 128)),
                   _f32((NROW, 128)), _f32((NROW, 128))],
    )(hagg, x, bias, wih, whh, wn, asrc, adst)


def _tcfinal_body(hagg_ref, x_ref, bias_ref, wih_ref, whh_ref,
                  wn_ref, asrc_ref, batch_ref,
                  xsrc_ref, as_ref, pool_ref):
    i = pl.program_id(0)
    hb = hagg_ref[...]
    den = _lb(jnp.maximum(hb[:, 64:65], 1e-30), 64)
    h = _elu(hb[:, 0:64] / den + bias_ref[...])
    xn = jnp.maximum(_gru(h, x_ref[...], wih_ref[...], whh_ref[...]), 0.0)
    xp = _dotT(xn, wn_ref[...])
    xsrc_ref[...] = xp
    as_ref[...] = _col2plane(_rowsum(xp * asrc_ref[...]))
    bc = _lb(_plane2col(batch_ref[...]), 128)
    oh = (bc == lax.broadcasted_iota(jnp.int32, (1024, 128), 1
          ).astype(jnp.float32)).astype(jnp.float32)
    part = lax.dot_general(oh, xn, (((0,), (0,)), ((), ())),
                           preferred_element_type=jnp.float32)

    @pl.when(i == 0)
    def _():
        pool_ref[...] = jnp.zeros_like(pool_ref)

    pool_ref[...] += part


def _tcfinal(hagg, x, bias, wih, whh, wn, asrc, batch2d):
    return pl.pallas_call(
        _tcfinal_body,
        grid=(49,),
        in_specs=[
            pl.BlockSpec((1024, 128), lambda i: (i, 0)),
            pl.BlockSpec((1024, 64), lambda i: (i, 0)),
            pl.BlockSpec((1, 64), lambda i: (0, 0)),
            pl.BlockSpec((192, 64), lambda i: (0, 0)),
            pl.BlockSpec((192, 64), lambda i: (0, 0)),
            pl.BlockSpec((64, 64), lambda i: (0, 0)),
            pl.BlockSpec((1, 64), lambda i: (0, 0)),
            pl.BlockSpec((8, 128), lambda i: (i, 0)),
        ],
        out_specs=[
            pl.BlockSpec((1024, 64), lambda i: (i, 0)),
            pl.BlockSpec((8, 128), lambda i: (i, 0)),
            pl.BlockSpec((128, 64), lambda i: (0, 0)),
        ],
        out_shape=[_f32((NPAD, 64)), _f32((NROW, 128)), _f32((128, 64))],
    )(hagg, x, bias, wih, whh, wn, asrc, batch2d)


def _r1b_body(pool_ref, asrc_ref, w_ref, adst_ref,
              out_ref, q_ref, sh_ref, am_ref):
    out0 = jnp.maximum(pool_ref[...], 0.0)
    out_ref[...] = out0
    op = _dotT(out0, w_ref[...])
    q = lax.dot_general(adst_ref[...], op, (((1,), (1,)), ((), ())),
                        preferred_element_type=jnp.float32)
    q_ref[...] = q
    amax = jnp.max(asrc_ref[...])
    am_ref[...] = jnp.reshape(amax, (1, 1))
    sh_ref[...] = jnp.reshape(jnp.maximum(amax + jnp.max(q), 0.0), (1, 1))


def _r1b(pool, asrc, w, adst):
    return pl.pallas_call(
        _r1b_body,
        out_shape=[_f32((128, 64)), _f32((1, 128)), _f32((1, 1)),
                   _f32((1, 1))],
    )(pool, asrc, w, adst)


def _r2b_body(xsrc_ref, asrc_ref, batch_ref, q_ref, sh_ref,
              num_ref, den_ref):
    i = pl.program_id(0)
    bc = _lb(_plane2col(batch_ref[...]), 128)
    oh = (bc == lax.broadcasted_iota(jnp.int32, (1024, 128), 1
          ).astype(jnp.float32)).astype(jnp.float32)
    qn = lax.dot_general(oh, q_ref[...], (((1,), (1,)), ((), ())),
                         preferred_element_type=jnp.float32)
    a = _leaky(_plane2col(asrc_ref[...]) + qn)
    e64 = _lb(jnp.exp(a - sh_ref[0, 0]), 64)
    num = lax.dot_general(oh, xsrc_ref[...] * e64, (((0,), (0,)), ((), ())),
                          preferred_element_type=jnp.float32)
    den = lax.dot_general(oh, e64, (((0,), (0,)), ((), ())),
                          preferred_element_type=jnp.float32)

    @pl.when(i == 0)
    def _():
        num_ref[...] = jnp.zeros_like(num_ref)
        den_ref[...] = jnp.zeros_like(den_ref)

    num_ref[...] += num
    den_ref[...] += den


def _r2b(xsrc, asrc, batch2d, q, sh):
    return pl.pallas_call(
        _r2b_body,
        grid=(49,),
        in_specs=[
            pl.BlockSpec((1024, 64), lambda i: (i, 0)),
            pl.BlockSpec((8, 128), lambda i: (i, 0)),
            pl.BlockSpec((8, 128), lambda i: (i, 0)),
            pl.BlockSpec((1, 128), lambda i: (0, 0)),
            pl.BlockSpec(memory_space=pltpu.SMEM),
        ],
        out_specs=[
            pl.BlockSpec((128, 64), lambda i: (0, 0)),
            pl.BlockSpec((128, 64), lambda i: (0, 0)),
        ],
        out_shape=[_f32((128, 64)), _f32((128, 64))],
    )(xsrc, asrc, batch2d, q, sh)


def _r2c_body(num_ref, den_ref, bias_ref, out_ref, wih_ref, whh_ref,
              w_ref, adst_ref, amaxs_ref,
              outn_ref, q_ref, sh_ref):
    h = _elu(num_ref[...] / jnp.maximum(den_ref[...], 1e-30) + bias_ref[...])
    outn = jnp.maximum(_gru(h, out_ref[...], wih_ref[...], whh_ref[...]), 0.0)
    outn_ref[...] = outn
    op = _dotT(outn, w_ref[...])
    q = lax.dot_general(adst_ref[...], op, (((1,), (1,)), ((), ())),
                        preferred_element_type=jnp.float32)
    q_ref[...] = q
    sh_ref[...] = jnp.reshape(
        jnp.maximum(amaxs_ref[0, 0] + jnp.max(q), 0.0), (1, 1))


def _r2c(num, den, bias, out, wih, whh, w, adst, amaxs):
    ms = pl.BlockSpec(memory_space=pltpu.SMEM)
    vs = pl.BlockSpec()
    return pl.pallas_call(
        _r2c_body,
        in_specs=[vs, vs, vs, vs, vs, vs, vs, vs, ms],
        out_shape=[_f32((128, 64)), _f32((1, 128)), _f32((1, 1))],
    )(num, den, bias, out, wih, whh, w, adst, amaxs)


def _r3_body(num_ref, den_ref, bias_ref, out_ref, wih_ref, whh_ref,
             w2_ref, b2_ref, res_ref):
    h = _elu(num_ref[...] / jnp.maximum(den_ref[...], 1e-30) + bias_ref[...])
    outn = jnp.maximum(_gru(h, out_ref[...], wih_ref[...], whh_ref[...]), 0.0)
    res_ref[...] = lax.dot_general(
        w2_ref[...], outn, (((1,), (1,)), ((), ())),
        preferred_element_type=jnp.float32) + b2_ref[0, 0]


def _r3(num, den, bias, out, wih, whh, w2, b2):
    ms = pl.BlockSpec(memory_space=pltpu.SMEM)
    vs = pl.BlockSpec()
    return pl.pallas_call(
        _r3_body,
        in_specs=[vs, vs, vs, vs, vs, vs, vs, ms],
        out_shape=_f32((1, 128)),
    )(num, den, bias, out, wih, whh, w2, b2)


def kernel(x, edge_index, edge_attr, batch, params):
    p = params
    src = edge_index[0]
    dst = edge_index[1]
    src_pad = jnp.concatenate([src, jnp.zeros((EPAD - E,), jnp.int32)])
    dst_pad = jnp.concatenate(
        [dst, jnp.full((EPAD - E,), 10_000_000, jnp.int32)])
    eaT = [jnp.concatenate([edge_attr[:, k],
                            jnp.zeros((EPAD - E,), jnp.float32)])
           for k in range(4)]
    xp_in = jnp.pad(x, ((0, NPAD - N), (0, 32 - x.shape[1])))
    batch2d = jnp.pad(batch, (0, NPAD - N), constant_values=NUM_GRAPHS
                      ).reshape(NROW, 128).astype(jnp.float32)
    zq = jnp.zeros((RPSQ, 128), jnp.float32)

    srcp, dstp, cnts = _partition(src_pad, dst_pad)

    w1p = jnp.pad(p["lin1_w"], ((0, 0), (0, 7)))
    wa = p["gate_lin1_w"][:, :HID]
    w2flat = p["gate_lin1_w"][:, HID:].T.reshape(-1)
    x1, u, m, t2 = _tc0(xp_in, w1p, p["lin1_b"][None, :], wa,
                        p["gate_lin2_w"], p["gate_att_r"][None, :])

    def message_layer(mtab, a_lin, amax):
        ap = _apart(a_lin, dst_pad)
        ha = _message0(mtab, ap, amax, srcp, dstp, cnts, zq)
        hb = _message1(mtab, ap, amax, srcp, dstp, cnts, zq)
        return jnp.concatenate([ha, hb], axis=0)

    a_lin, amax = _galpha(u, t2.reshape(-1), w2flat, p["gate_att_l"],
                          src_pad, dst_pad, eaT[0], eaT[1], eaT[2], eaT[3])
    hagg = message_layer(m, a_lin, amax)

    cur_x = x1
    bias = p["gate_bias"][None, :]
    gru = p["gru0"]
    for li in range(4):
        conv = p["atom_convs"][li]
        cur_x, xp128, s2, d2 = _tclayer(
            hagg, cur_x, bias, gru["wih"], gru["whh"],
            conv["w"], conv["att_src"][None, :], conv["att_dst"][None, :])
        a_lin, amax = _alpha(s2.reshape(-1), d2.reshape(-1),
                             src_pad, dst_pad)
        hagg = message_layer(xp128, a_lin, amax)
        bias = conv["bias"][None, :]
        gru = p["atom_grus"][li]

    mc, mg = p["mol_conv"], p["mol_gru"]
    xsrc, asrc2, pool = _tcfinal(
        hagg, cur_x, bias, gru["wih"], gru["whh"],
        mc["w"], mc["att_src"][None, :], batch2d)

    out0, q, sh, amaxs = _r1b(pool, asrc2, mc["w"], mc["att_dst"][None, :])
    outc = out0
    bias_m = mc["bias"][None, :]
    for ts in range(3):
        num, den_g = _r2b(xsrc, asrc2, batch2d, q, sh)
        if ts < 2:
            outc, q, sh = _r2c(num, den_g, bias_m, outc,
                               mg["wih"], mg["whh"], mc["w"],
                               mc["att_dst"][None, :], amaxs)
        else:
            res = _r3(num, den_g, bias_m, outc, mg["wih"], mg["whh"],
                      p["lin2_w"], p["lin2_b"][None, :])
    return res.reshape(-1)


# speedup vs baseline: 1.3863x; 1.3863x over previous
import jax
import jax.numpy as jnp
from jax import lax
from jax.experimental import pallas as pl
from jax.experimental.pallas import tpu as pltpu
from jax.experimental.pallas import tpu_sc as plsc

N = 50000
E = 800000
HID = 64
NUM_GRAPHS = 128

NPAD = 50176
NROW = NPAD // 128
NT = 50304
Q = 12544
ACCQ = 12552
RPSQ = Q // 16
EPW = 25088
EPAD = 32 * EPW
CAPQ = 9216
BLK = 512
MBLK = 176
MBAT = 1408
NSLQ = 128
LEAK = 0.01
NEG = -1.0e30

_mesh = plsc.VectorSubcoreMesh(core_axis_name="c", subcore_axis_name="s")
_sc_params = pltpu.CompilerParams(needs_layout_passes=False)


def _f32(shape):
    return jax.ShapeDtypeStruct(shape, jnp.float32)


def _i32(shape):
    return jax.ShapeDtypeStruct(shape, jnp.int32)


def _leaky(v):
    return jnp.where(v >= 0, v, LEAK * v)


def _lane_splat(v, j):
    idx = jnp.full((16, 1), j, dtype=jnp.int32)
    dnums = lax.GatherDimensionNumbers(
        offset_dims=(), collapsed_slice_dims=(0,), start_index_map=(0,))
    return lax.gather(v, idx, dnums, (1,),
                      mode=lax.GatherScatterMode.PROMISE_IN_BOUNDS)


def _scnt(mask):
    return jnp.max(plsc.all_reduce_population_count(mask))


def _qrank(cq, m):
    rank = plsc.cumsum(jnp.where(m, 1.0, 0.0)).astype(jnp.int32)
    return cq + rank - 1


def _part_body(src_h, dst_h, srcp_h, dstp_h, cnt_h,
               src_c, dst_c, sl0, dl0, sl1, dl1, sl2, dl2, sl3, dl3, cntv):
    wid = lax.axis_index("s") * 2 + lax.axis_index("c")
    base = wid * EPW
    lanes = lax.iota(jnp.int32, 16)
    sls = (sl0, sl1, sl2, sl3)
    dls = (dl0, dl1, dl2, dl3)

    def chunk(co, nvr, cnts):
        pltpu.sync_copy(src_h.at[pl.ds(base + co, nvr * 16)],
                        src_c.at[pl.ds(0, nvr * 16)])
        pltpu.sync_copy(dst_h.at[pl.ds(base + co, nvr * 16)],
                        dst_c.at[pl.ds(0, nvr * 16)])

        def vbody(v, cnts):
            sv = src_c[pl.ds(v * 16, 16)]
            dv = dst_c[pl.ds(v * 16, 16)]
            out = []
            for q in range(4):
                m = jnp.logical_and(dv >= q * Q, dv < (q + 1) * Q)
                iq = _qrank(cnts[q], m)
                plsc.store_scatter(sls[q], [iq], sv, mask=m)
                plsc.store_scatter(dls[q], [iq], dv - q * Q, mask=m)
                out.append(cnts[q] + _scnt(m))
            return tuple(out)

        return lax.fori_loop(0, nvr, vbody, cnts)

    cnts = (jnp.int32(0),) * 4
    def outer(k, cnts):
        return chunk(k * 4096, 256, cnts)
    cnts = lax.fori_loop(0, 6, outer, cnts)
    cnts = chunk(6 * 4096, 32, cnts)

    dumdst = Q + (lanes & 7)
    zsrc = jnp.zeros((16,), jnp.int32)
    for q in range(4):
        for i in range(32):
            sls[q][pl.ds(cnts[q] + i * 16, 16)] = zsrc
            dls[q][pl.ds(cnts[q] + i * 16, 16)] = dumdst

    for q in range(4):
        off = (q * 32 + wid) * CAPQ
        pltpu.sync_copy(sls[q], srcp_h.at[pl.ds(off, CAPQ)])
        pltpu.sync_copy(dls[q], dstp_h.at[pl.ds(off, CAPQ)])
        cntv[...] = jnp.broadcast_to(cnts[q], (16,)).astype(jnp.int32)
        pltpu.sync_copy(cntv, cnt_h.at[q * 32 + wid])


_partition = pl.kernel(
    _part_body,
    out_type=(_i32((NSLQ * CAPQ,)), _i32((NSLQ * CAPQ,)), _i32((NSLQ, 16))),
    mesh=_mesh,
    compiler_params=_sc_params,
    scratch_types=[
        pltpu.VMEM((4096,), jnp.int32), pltpu.VMEM((4096,), jnp.int32),
        pltpu.VMEM((CAPQ,), jnp.int32), pltpu.VMEM((CAPQ,), jnp.int32),
        pltpu.VMEM((CAPQ,), jnp.int32), pltpu.VMEM((CAPQ,), jnp.int32),
        pltpu.VMEM((CAPQ,), jnp.int32), pltpu.VMEM((CAPQ,), jnp.int32),
        pltpu.VMEM((CAPQ,), jnp.int32), pltpu.VMEM((CAPQ,), jnp.int32),
        pltpu.VMEM((16,), jnp.int32),
    ],
)


def _apart_body(a_h, dst_h, ap_h,
                a_c, dst_c, al0, al1, al2, al3):
    wid = lax.axis_index("s") * 2 + lax.axis_index("c")
    base = wid * EPW
    als = (al0, al1, al2, al3)

    def chunk(co, nvr, cnts):
        pltpu.sync_copy(a_h.at[pl.ds(base + co, nvr * 16)],
                        a_c.at[pl.ds(0, nvr * 16)])
        pltpu.sync_copy(dst_h.at[pl.ds(base + co, nvr * 16)],
                        dst_c.at[pl.ds(0, nvr * 16)])

        def vbody(v, cnts):
            av = a_c[pl.ds(v * 16, 16)]
            dv = dst_c[pl.ds(v * 16, 16)]
            out = []
            for q in range(4):
                m = jnp.logical_and(dv >= q * Q, dv < (q + 1) * Q)
                iq = _qrank(cnts[q], m)
                plsc.store_scatter(als[q], [iq], av, mask=m)
                out.append(cnts[q] + _scnt(m))
            return tuple(out)

        return lax.fori_loop(0, nvr, vbody, cnts)

    cnts = (jnp.int32(0),) * 4
    def outer(k, cnts):
        return chunk(k * 4096, 256, cnts)
    cnts = lax.fori_loop(0, 6, outer, cnts)
    cnts = chunk(6 * 4096, 32, cnts)

    negv = jnp.full((16,), NEG, jnp.float32)
    for q in range(4):
        for i in range(32):
            als[q][pl.ds(cnts[q] + i * 16, 16)] = negv

    for q in range(4):
        off = (q * 32 + wid) * CAPQ
        pltpu.sync_copy(als[q], ap_h.at[pl.ds(off, CAPQ)])


_apart = pl.kernel(
    _apart_body,
    out_type=_f32((NSLQ * CAPQ,)),
    mesh=_mesh,
    compiler_params=_sc_params,
    scratch_types=[
        pltpu.VMEM((4096,), jnp.float32), pltpu.VMEM((4096,), jnp.int32),
        pltpu.VMEM((CAPQ,), jnp.float32), pltpu.VMEM((CAPQ,), jnp.float32),
        pltpu.VMEM((CAPQ,), jnp.float32), pltpu.VMEM((CAPQ,), jnp.float32),
    ],
)


def _alpha_body(s_h, d_h, src_h, dst_h,
                a_h, amax_h,
                stab, dtab, src_c, dst_c, a_c, maxv):
    wid = lax.axis_index("s") * 2 + lax.axis_index("c")
    base = wid * EPW
    pltpu.sync_copy(s_h, stab.at[pl.ds(0, NPAD)])
    pltpu.sync_copy(d_h, dtab.at[pl.ds(0, NPAD)])
    zf = jnp.zeros((16,), jnp.float32)
    for k in range((NT - NPAD) // 16):
        stab[pl.ds(NPAD + k * 16, 16)] = zf
        dtab[pl.ds(NPAD + k * 16, 16)] = zf

    def chunk(co, nvr, mx):
        pltpu.sync_copy(src_h.at[pl.ds(base + co, nvr * 16)],
                        src_c.at[pl.ds(0, nvr * 16)])
        pltpu.sync_copy(dst_h.at[pl.ds(base + co, nvr * 16)],
                        dst_c.at[pl.ds(0, nvr * 16)])

        def vbody(v, mx):
            sv = src_c[pl.ds(v * 16, 16)]
            dv = dst_c[pl.ds(v * 16, 16)]
            dcl = jnp.minimum(dv, NT - 1)
            a = _leaky(plsc.load_gather(stab, [sv])
                       + plsc.load_gather(dtab, [dcl]))
            a_c[pl.ds(v * 16, 16)] = a
            return jnp.maximum(mx, jnp.where(dv < NPAD, a, NEG))

        mx = lax.fori_loop(0, nvr, vbody, mx)
        pltpu.sync_copy(a_c.at[pl.ds(0, nvr * 16)],
                        a_h.at[pl.ds(base + co, nvr * 16)])
        return mx

    mx = jnp.full((16,), NEG, jnp.float32)
    def outer(k, mx):
        return chunk(k * 4096, 256, mx)
    mx = lax.fori_loop(0, 6, outer, mx)
    mx = chunk(6 * 4096, 32, mx)
    maxv[...] = mx
    pltpu.sync_copy(maxv, amax_h.at[wid])


_alpha = pl.kernel(
    _alpha_body,
    out_type=(_f32((EPAD,)), _f32((32, 16))),
    mesh=_mesh,
    compiler_params=_sc_params,
    scratch_types=[
        pltpu.VMEM((NT,), jnp.float32), pltpu.VMEM((NT,), jnp.float32),
        pltpu.VMEM((4096,), jnp.int32), pltpu.VMEM((4096,), jnp.int32),
        pltpu.VMEM((4096,), jnp.float32), pltpu.VMEM((16,), jnp.float32),
    ],
)


def _galpha_body(u_h, t_h, w2_h, attl_h, src_h, dst_h,
                 ea0_h, ea1_h, ea2_h, ea3_h,
                 a_h, amax_h,
                 ttab, urows, src_c, dst_c, a_c,
                 e0c, e1c, e2c, e3c, w2v, attlv, maxv, sem):
    wid = lax.axis_index("s") * 2 + lax.axis_index("c")
    base = wid * EPW
    pltpu.sync_copy(t_h, ttab.at[pl.ds(0, NPAD)])
    zf = jnp.zeros((16,), jnp.float32)
    for k in range((NT - NPAD) // 16):
        ttab[pl.ds(NPAD + k * 16, 16)] = zf
    pltpu.sync_copy(w2_h, w2v)
    pltpu.sync_copy(attl_h, attlv)
    w2 = [[w2v[pl.ds(k * 64 + c * 16, 16)] for c in range(4)]
          for k in range(4)]
    attl = [attlv[pl.ds(c * 16, 16)] for c in range(4)]
    eacs = (e0c, e1c, e2c, e3c)
    eahs = (ea0_h, ea1_h, ea2_h, ea3_h)

    def blk(b, mx):
        off = base + b * BLK
        pltpu.sync_copy(src_h.at[pl.ds(off, BLK)], src_c)
        pltpu.sync_copy(dst_h.at[pl.ds(off, BLK)], dst_c)
        for k in range(4):
            pltpu.sync_copy(eahs[k].at[pl.ds(off, BLK)], eacs[k])
        pltpu.async_copy(u_h.at[src_c], urows, sem).wait()

        def vbody(v, mx):
            phi = jnp.zeros((16,), jnp.float32)
            eav = [eacs[k][pl.ds(v * 16, 16)] for k in range(4)]
            for j in range(16):
                e = v * 16 + j
                eas = [_lane_splat(eav[k], j) for k in range(4)]
                acc = jnp.zeros((16,), jnp.float32)
                for c in range(4):
                    z = urows[e, pl.ds(c * 16, 16)]
                    for k in range(4):
                        z = z + eas[k] * w2[k][c]
                    acc = acc + _leaky(z) * attl[c]
                sjs = jnp.sum(acc)
                phi = jnp.where(lax.iota(jnp.int32, 16) == j, sjs, phi)
            dv = dst_c[pl.ds(v * 16, 16)]
            dcl = jnp.minimum(dv, NT - 1)
            a = _leaky(phi + plsc.load_gather(ttab, [dcl]))
            a_c[pl.ds(v * 16, 16)] = a
            return jnp.maximum(mx, jnp.where(dv < NPAD, a, NEG))

        mx = lax.fori_loop(0, BLK // 16, vbody, mx)
        pltpu.sync_copy(a_c, a_h.at[pl.ds(off, BLK)])
        return mx

    mx = jnp.full((16,), NEG, jnp.float32)
    mx = lax.fori_loop(0, EPW // BLK, blk, mx)
    maxv[...] = mx
    pltpu.sync_copy(maxv, amax_h.at[wid])


_galpha = pl.kernel(
    _galpha_body,
    out_type=(_f32((EPAD,)), _f32((32, 16))),
    mesh=_mesh,
    compiler_params=_sc_params,
    scratch_types=[
        pltpu.VMEM((NT,), jnp.float32),
        pltpu.VMEM((BLK, 128), jnp.float32),
        pltpu.VMEM((BLK,), jnp.int32), pltpu.VMEM((BLK,), jnp.int32),
        pltpu.VMEM((BLK,), jnp.float32),
        pltpu.VMEM((BLK,), jnp.float32), pltpu.VMEM((BLK,), jnp.float32),
        pltpu.VMEM((BLK,), jnp.float32), pltpu.VMEM((BLK,), jnp.float32),
        pltpu.VMEM((256,), jnp.float32), pltpu.VMEM((64,), jnp.float32),
        pltpu.VMEM((16,), jnp.float32),
        pltpu.SemaphoreType.DMA,
    ],
)


def _make_message(pair):

    def body(m_h, ap_h, amax_h, srcp_h, dstp_h, cnt_h, z_h,
             hagg_h,
             rows, sbat, abat, dblk, amv, cntv, sem, acc_sh):
        c = lax.axis_index("c")
        s = lax.axis_index("s")
        qrow = (2 * pair + c) * 32

        pltpu.sync_copy(z_h.at[pl.ds(0, RPSQ)],
                        acc_sh.at[pl.ds(s * RPSQ, RPSQ)])

        @pl.when(s == 15)
        def _():
            pltpu.sync_copy(z_h.at[pl.ds(0, 8)], acc_sh.at[pl.ds(Q, 8)])

        pltpu.sync_copy(amax_h, amv)
        gv = jnp.full((16,), NEG, jnp.float32)
        for w in range(32):
            gv = jnp.maximum(gv, amv[w])
        gmax = jnp.max(gv)
        colm = (lax.iota(jnp.int32, 16) == 0).astype(jnp.float32)

        plsc.subcore_barrier()

        def do_slot(w):
            row = qrow + w
            cnt = _read_cnt(cnt_h, row, cntv)
            nblk = (cnt + (MBLK - 1)) // MBLK
            nbat = (nblk + 7) >> 3
            base = row * CAPQ

            def bat_body(t, carry):
                boff = base + t * MBAT
                pltpu.sync_copy(srcp_h.at[pl.ds(boff, MBAT)], sbat)
                pltpu.sync_copy(ap_h.at[pl.ds(boff, MBAT)], abat)
                kb = jnp.minimum(8, nblk - t * 8)

                def blk_body(k, carry2):
                    off = boff + k * MBLK
                    pltpu.sync_copy(dstp_h.at[pl.ds(off, MBLK)], dblk)
                    pltpu.async_copy(
                        m_h.at[sbat.at[pl.ds(k * MBLK, MBLK)]], rows,
                        sem).wait()

                    def vbody(v, carry3):
                        ex = jnp.exp(
                            abat[pl.ds(k * MBLK + v * 16, 16)] - gmax)
                        for j in range(16):
                            wj = _lane_splat(ex, j)
                            e = v * 16 + j
                            for ch in range(4):
                                rows[e, pl.ds(ch * 16, 16)] = (
                                    rows[e, pl.ds(ch * 16, 16)] * wj)
                            rows[e, pl.ds(64, 16)] = wj * colm
                        return carry3

                    lax.fori_loop(0, MBLK // 16, vbody, 0)
                    pltpu.sync_copy(rows, acc_sh.at[dblk], add=True)
                    return carry2

                lax.fori_loop(0, kb, blk_body, 0)
                return carry

            lax.fori_loop(0, nbat, bat_body, 0)

        do_slot(2 * s)
        do_slot(2 * s + 1)

        plsc.subcore_barrier()
        pltpu.sync_copy(acc_sh.at[pl.ds(s * RPSQ, RPSQ)],
                        hagg_h.at[pl.ds(c * Q + s * RPSQ, RPSQ)])

    return pl.kernel(
        body,
        out_type=_f32((2 * Q, 128)),
        mesh=_mesh,
        compiler_params=_sc_params,
        scratch_types=[
            pltpu.VMEM((MBLK, 128), jnp.float32),
            pltpu.VMEM((MBAT,), jnp.int32),
            pltpu.VMEM((MBAT,), jnp.float32),
            pltpu.VMEM((MBLK,), jnp.int32),
            pltpu.VMEM((32, 16), jnp.float32), pltpu.VMEM((16,), jnp.int32),
            pltpu.SemaphoreType.DMA,
            pltpu.VMEM_SHARED((ACCQ, 128), jnp.float32),
        ],
    )


_message0 = _make_message(0)
_message1 = _make_message(1)


def _read_cnt(cnt_h, row, cntv):
    pltpu.sync_copy(cnt_h.at[row], cntv)
    return jnp.max(cntv[...])


def _dotT(a, b):
    return lax.dot_general(a, b, (((1,), (1,)), ((), ())),
                           preferred_element_type=jnp.float32)


def _rowsum(x):
    return jnp.sum(x, axis=1, keepdims=True)


def _s1mat():
    r = lax.broadcasted_iota(jnp.int32, (8, 1024), 0)
    n = lax.broadcasted_iota(jnp.int32, (8, 1024), 1)
    return (r == (n >> 7)).astype(jnp.float32)


def _m128mat():
    n = lax.broadcasted_iota(jnp.int32, (1024, 128), 0)
    c = lax.broadcasted_iota(jnp.int32, (1024, 128), 1)
    return ((n & 127) == c).astype(jnp.float32)


def _col2plane(tcol):
    return lax.dot_general(_s1mat(), _lb(tcol, 128) * _m128mat(),
                           (((1,), (0,)), ((), ())),
                           preferred_element_type=jnp.float32)


def _lb(col, n):
    return lax.dot_general(col, jnp.ones((1, n), jnp.float32),
                           (((1,), (0,)), ((), ())),
                           preferred_element_type=jnp.float32)


def _plane2col(p8):
    y = lax.dot_general(_s1mat(), p8, (((0,), (0,)), ((), ())),
                        preferred_element_type=jnp.float32)
    return _rowsum(y * _m128mat())


def _gru(h, x, wih, whh):
    gi = _dotT(h, wih)
    gh = _dotT(x, whh)
    r = jax.nn.sigmoid(gi[:, 0:64] + gh[:, 0:64])
    z = jax.nn.sigmoid(gi[:, 64:128] + gh[:, 64:128])
    n = jnp.tanh(gi[:, 128:192] + r * gh[:, 128:192])
    return (1.0 - z) * n + z * x


def _elu(v):
    return jnp.where(v > 0, v, jnp.exp(v) - 1.0)


def _tc0_body(x_ref, w1_ref, b1_ref, wa_ref, w2g_ref, attr_ref,
              x1_ref, u_ref, m_ref, t_ref):
    i = pl.program_id(0)
    x1 = _leaky(_dotT(x_ref[...], w1_ref[...]) + b1_ref[...])
    ridx = i * 1024 + lax.broadcasted_iota(jnp.int32, (1024, 64), 0)
    x1 = jnp.where(ridx < N, x1, 0.0)
    x1_ref[...] = x1
    z = jnp.zeros((1024, 64), jnp.float32)
    u_ref[...] = jnp.concatenate([_dotT(x1, wa_ref[...]), z], axis=1)
    m_ref[...] = jnp.concatenate([_dotT(x1, w2g_ref[...]), z], axis=1)
    t_ref[...] = _col2plane(_rowsum(x1 * attr_ref[...]))


def _tc0(xp, w1p, b1, wa, w2g, attr):
    return pl.pallas_call(
        _tc0_body,
        grid=(49,),
        in_specs=[
            pl.BlockSpec((1024, 32), lambda i: (i, 0)),
            pl.BlockSpec((64, 32), lambda i: (0, 0)),
            pl.BlockSpec((1, 64), lambda i: (0, 0)),
            pl.BlockSpec((64, 64), lambda i: (0, 0)),
            pl.BlockSpec((64, 64), lambda i: (0, 0)),
            pl.BlockSpec((1, 64), lambda i: (0, 0)),
        ],
        out_specs=[
            pl.BlockSpec((1024, 64), lambda i: (i, 0)),
            pl.BlockSpec((1024, 128), lambda i: (i, 0)),
            pl.BlockSpec((1024, 128), lambda i: (i, 0)),
            pl.BlockSpec((8, 128), lambda i: (i, 0)),
        ],
        out_shape=[_f32((NPAD, 64)), _f32((NPAD, 128)), _f32((NPAD, 128)),
                   _f32((NROW, 128))],
    )(xp, w1p, b1, wa, w2g, attr)


def _tclayer_body(hagg_ref, x_ref, bias_ref, wih_ref, whh_ref,
                  wn_ref, asrc_ref, adst_ref,
                  xn_ref, xp_ref, s_ref, d_ref):
    hb = hagg_ref[...]
    den = _lb(jnp.maximum(hb[:, 64:65], 1e-30), 64)
    h = _elu(hb[:, 0:64] / den + bias_ref[...])
    xn = jnp.maximum(_gru(h, x_ref[...], wih_ref[...], whh_ref[...]), 0.0)
    xn_ref[...] = xn
    xp = _dotT(xn, wn_ref[...])
    xp_ref[...] = jnp.concatenate(
        [xp, jnp.zeros((1024, 64), jnp.float32)], axis=1)
    s_ref[...] = _col2plane(_rowsum(xp * asrc_ref[...]))
    d_ref[...] = _col2plane(_rowsum(xp * adst_ref[...]))


def _tclayer(hagg, x, bias, wih, whh, wn, asrc, adst):
    return pl.pallas_call(
        _tclayer_body,
        grid=(49,),
        in_specs=[
            pl.BlockSpec((1024, 128), lambda i: (i, 0)),
            pl.BlockSpec((1024, 64), lambda i: (i, 0)),
            pl.BlockSpec((1, 64), lambda i: (0, 0)),
            pl.BlockSpec((192, 64), lambda i: (0, 0)),
            pl.BlockSpec((192, 64), lambda i: (0, 0)),
            pl.BlockSpec((64, 64), lambda i: (0, 0)),
            pl.BlockSpec((1, 64), lambda i: (0, 0)),
            pl.BlockSpec((1, 64), lambda i: (0, 0)),
        ],
        out_specs=[
            pl.BlockSpec((1024, 64), lambda i: (i, 0)),
            pl.BlockSpec((1024, 128), lambda i: (i, 0)),
            pl.BlockSpec((8, 128), lambda i: (i, 0)),
            pl.BlockSpec((8, 128), lambda i: (i, 0)),
        ],
        out_shape=[_f32((NPAD, 64)), _f32((NPAD, 128)),
                   _f32((NROW, 128)), _f32((NROW, 128))],
    )(hagg, x, bias, wih, whh, wn, asrc, adst)


def _tcfinal_body(hagg_ref, x_ref, bias_ref, wih_ref, whh_ref,
                  wn_ref, asrc_ref, batch_ref,
                  xsrc_ref, as_ref, pool_ref):
    i = pl.program_id(0)
    hb = hagg_ref[...]
    den = _lb(jnp.maximum(hb[:, 64:65], 1e-30), 64)
    h = _elu(hb[:, 0:64] / den + bias_ref[...])
    xn = jnp.maximum(_gru(h, x_ref[...], wih_ref[...], whh_ref[...]), 0.0)
    xp = _dotT(xn, wn_ref[...])
    xsrc_ref[...] = xp
    as_ref[...] = _col2plane(_rowsum(xp * asrc_ref[...]))
    bc = _lb(_plane2col(batch_ref[...]), 128)
    oh = (bc == lax.broadcasted_iota(jnp.int32, (1024, 128), 1
          ).astype(jnp.float32)).astype(jnp.float32)
    part = lax.dot_general(oh, xn, (((0,), (0,)), ((), ())),
                           preferred_element_type=jnp.float32)

    @pl.when(i == 0)
    def _():
        pool_ref[...] = jnp.zeros_like(pool_ref)

    pool_ref[...] += part


def _tcfinal(hagg, x, bias, wih, whh, wn, asrc, batch2d):
    return pl.pallas_call(
        _tcfinal_body,
        grid=(49,),
        in_specs=[
            pl.BlockSpec((1024, 128), lambda i: (i, 0)),
            pl.BlockSpec((1024, 64), lambda i: (i, 0)),
            pl.BlockSpec((1, 64), lambda i: (0, 0)),
            pl.BlockSpec((192, 64), lambda i: (0, 0)),
            pl.BlockSpec((192, 64), lambda i: (0, 0)),
            pl.BlockSpec((64, 64), lambda i: (0, 0)),
            pl.BlockSpec((1, 64), lambda i: (0, 0)),
            pl.BlockSpec((8, 128), lambda i: (i, 0)),
        ],
        out_specs=[
            pl.BlockSpec((1024, 64), lambda i: (i, 0)),
            pl.BlockSpec((8, 128), lambda i: (i, 0)),
            pl.BlockSpec((128, 64), lambda i: (0, 0)),
        ],
        out_shape=[_f32((NPAD, 64)), _f32((NROW, 128)), _f32((128, 64))],
    )(hagg, x, bias, wih, whh, wn, asrc, batch2d)


def _r1b_body(pool_ref, asrc_ref, w_ref, adst_ref,
              out_ref, q_ref, sh_ref, am_ref):
    out0 = jnp.maximum(pool_ref[...], 0.0)
    out_ref[...] = out0
    op = _dotT(out0, w_ref[...])
    q = lax.dot_general(adst_ref[...], op, (((1,), (1,)), ((), ())),
                        preferred_element_type=jnp.float32)
    q_ref[...] = q
    amax = jnp.max(asrc_ref[...])
    am_ref[...] = jnp.reshape(amax, (1, 1))
    sh_ref[...] = jnp.reshape(jnp.maximum(amax + jnp.max(q), 0.0), (1, 1))


def _r1b(pool, asrc, w, adst):
    return pl.pallas_call(
        _r1b_body,
        out_shape=[_f32((128, 64)), _f32((1, 128)), _f32((1, 1)),
                   _f32((1, 1))],
    )(pool, asrc, w, adst)


def _r2b_body(xsrc_ref, asrc_ref, batch_ref, q_ref, sh_ref,
              num_ref, den_ref):
    i = pl.program_id(0)
    bc = _lb(_plane2col(batch_ref[...]), 128)
    oh = (bc == lax.broadcasted_iota(jnp.int32, (1024, 128), 1
          ).astype(jnp.float32)).astype(jnp.float32)
    qn = lax.dot_general(oh, q_ref[...], (((1,), (1,)), ((), ())),
                         preferred_element_type=jnp.float32)
    a = _leaky(_plane2col(asrc_ref[...]) + qn)
    e64 = _lb(jnp.exp(a - sh_ref[0, 0]), 64)
    num = lax.dot_general(oh, xsrc_ref[...] * e64, (((0,), (0,)), ((), ())),
                          preferred_element_type=jnp.float32)
    den = lax.dot_general(oh, e64, (((0,), (0,)), ((), ())),
                          preferred_element_type=jnp.float32)

    @pl.when(i == 0)
    def _():
        num_ref[...] = jnp.zeros_like(num_ref)
        den_ref[...] = jnp.zeros_like(den_ref)

    num_ref[...] += num
    den_ref[...] += den


def _r2b(xsrc, asrc, batch2d, q, sh):
    return pl.pallas_call(
        _r2b_body,
        grid=(49,),
        in_specs=[
            pl.BlockSpec((1024, 64), lambda i: (i, 0)),
            pl.BlockSpec((8, 128), lambda i: (i, 0)),
            pl.BlockSpec((8, 128), lambda i: (i, 0)),
            pl.BlockSpec((1, 128), lambda i: (0, 0)),
            pl.BlockSpec(memory_space=pltpu.SMEM),
        ],
        out_specs=[
            pl.BlockSpec((128, 64), lambda i: (0, 0)),
            pl.BlockSpec((128, 64), lambda i: (0, 0)),
        ],
        out_shape=[_f32((128, 64)), _f32((128, 64))],
    )(xsrc, asrc, batch2d, q, sh)


def _r2c_body(num_ref, den_ref, bias_ref, out_ref, wih_ref, whh_ref,
              w_ref, adst_ref, amaxs_ref,
              outn_ref, q_ref, sh_ref):
    h = _elu(num_ref[...] / jnp.maximum(den_ref[...], 1e-30) + bias_ref[...])
    outn = jnp.maximum(_gru(h, out_ref[...], wih_ref[...], whh_ref[...]), 0.0)
    outn_ref[...] = outn
    op = _dotT(outn, w_ref[...])
    q = lax.dot_general(adst_ref[...], op, (((1,), (1,)), ((), ())),
                        preferred_element_type=jnp.float32)
    q_ref[...] = q
    sh_ref[...] = jnp.reshape(
        jnp.maximum(amaxs_ref[0, 0] + jnp.max(q), 0.0), (1, 1))


def _r2c(num, den, bias, out, wih, whh, w, adst, amaxs):
    ms = pl.BlockSpec(memory_space=pltpu.SMEM)
    vs = pl.BlockSpec()
    return pl.pallas_call(
        _r2c_body,
        in_specs=[vs, vs, vs, vs, vs, vs, vs, vs, ms],
        out_shape=[_f32((128, 64)), _f32((1, 128)), _f32((1, 1))],
    )(num, den, bias, out, wih, whh, w, adst, amaxs)


def _r3_body(num_ref, den_ref, bias_ref, out_ref, wih_ref, whh_ref,
             w2_ref, b2_ref, res_ref):
    h = _elu(num_ref[...] / jnp.maximum(den_ref[...], 1e-30) + bias_ref[...])
    outn = jnp.maximum(_gru(h, out_ref[...], wih_ref[...], whh_ref[...]), 0.0)
    res_ref[...] = lax.dot_general(
        w2_ref[...], outn, (((1,), (1,)), ((), ())),
        preferred_element_type=jnp.float32) + b2_ref[0, 0]


def _r3(num, den, bias, out, wih, whh, w2, b2):
    ms = pl.BlockSpec(memory_space=pltpu.SMEM)
    vs = pl.BlockSpec()
    return pl.pallas_call(
        _r3_body,
        in_specs=[vs, vs, vs, vs, vs, vs, vs, ms],
        out_shape=_f32((1, 128)),
    )(num, den, bias, out, wih, whh, w2, b2)


def kernel(x, edge_index, edge_attr, batch, params):
    p = params
    src = edge_index[0]
    dst = edge_index[1]
    src_pad = jnp.concatenate([src, jnp.zeros((EPAD - E,), jnp.int32)])
    dst_pad = jnp.concatenate(
        [dst, jnp.full((EPAD - E,), 10_000_000, jnp.int32)])
    eaT = [jnp.concatenate([edge_attr[:, k],
                            jnp.zeros((EPAD - E,), jnp.float32)])
           for k in range(4)]
    xp_in = jnp.pad(x, ((0, NPAD - N), (0, 32 - x.shape[1])))
    batch2d = jnp.pad(batch, (0, NPAD - N), constant_values=NUM_GRAPHS
                      ).reshape(NROW, 128).astype(jnp.float32)
    zq = jnp.zeros((RPSQ, 128), jnp.float32)

    srcp, dstp, cnts = _partition(src_pad, dst_pad)

    w1p = jnp.pad(p["lin1_w"], ((0, 0), (0, 7)))
    wa = p["gate_lin1_w"][:, :HID]
    w2flat = p["gate_lin1_w"][:, HID:].T.reshape(-1)
    x1, u, m, t2 = _tc0(xp_in, w1p, p["lin1_b"][None, :], wa,
                        p["gate_lin2_w"], p["gate_att_r"][None, :])

    def message_layer(mtab, a_lin, amax):
        ap = _apart(a_lin, dst_pad)
        ha = _message0(mtab, ap, amax, srcp, dstp, cnts, zq)
        hb = _message1(mtab, ap, amax, srcp, dstp, cnts, zq)
        return jnp.concatenate([ha, hb], axis=0)

    a_lin, amax = _galpha(u, t2.reshape(-1), w2flat, p["gate_att_l"],
                          src_pad, dst_pad, eaT[0], eaT[1], eaT[2], eaT[3])
    hagg = message_layer(m, a_lin, amax)

    cur_x = x1
    bias = p["gate_bias"][None, :]
    gru = p["gru0"]
    for li in range(4):
        conv = p["atom_convs"][li]
        cur_x, xp128, s2, d2 = _tclayer(
            hagg, cur_x, bias, gru["wih"], gru["whh"],
            conv["w"], conv["att_src"][None, :], conv["att_dst"][None, :])
        a_lin, amax = _alpha(s2.reshape(-1), d2.reshape(-1),
                             src_pad, dst_pad)
        hagg = message_layer(xp128, a_lin, amax)
        bias = conv["bias"][None, :]
        gru = p["atom_grus"][li]

    mc, mg = p["mol_conv"], p["mol_gru"]
    xsrc, asrc2, pool = _tcfinal(
        hagg, cur_x, bias, gru["wih"], gru["whh"],
        mc["w"], mc["att_src"][None, :], batch2d)

    out0, q, sh, amaxs = _r1b(pool, asrc2, mc["w"], mc["att_dst"][None, :])
    outc = out0
    bias_m = mc["bias"][None, :]
    for ts in range(3):
        num, den_g = _r2b(xsrc, asrc2, batch2d, q, sh)
        if ts < 2:
            outc, q, sh = _r2c(num, den_g, bias_m, outc,
                               mg["wih"], mg["whh"], mc["w"],
                               mc["att_dst"][None, :], amaxs)
        else:
            res = _r3(num, den_g, bias_m, outc, mg["wih"], mg["whh"],
                      p["lin2_w"], p["lin2_b"][None, :])
    return res.reshape(-1)
